# baseline (speedup 1.0000x reference)
import functools

import jax
import jax.numpy as jnp
from jax import lax
from jax.experimental import pallas as pl
from jax.experimental.pallas import tpu as pltpu
from jax.experimental.pallas import tpu_sc as plsc

N = 10000
E = 320000
D = 128
H = 128
K = 2

NC = 2
NS = 16
NW = NC * NS
CH = 128
NCHUNK = 80
EPW = NCHUNK * CH
E_PAD = NW * EPW
N_PAD = 10240
ZROWS = N_PAD // NS
R = 1000


def _seg_sum(table, src2d, dst2d, zrows):
    mesh = plsc.VectorSubcoreMesh(core_axis_name="c", subcore_axis_name="s")

    @functools.partial(
        pl.kernel,
        mesh=mesh,
        out_type=jax.ShapeDtypeStruct((NC, N_PAD, H), jnp.float32),
        scratch_types=[
            pltpu.VMEM((NCHUNK // 2, CH), jnp.int32),
            pltpu.VMEM((NCHUNK // 2, CH), jnp.int32),
            pltpu.VMEM((CH, H), jnp.float32),
            pltpu.VMEM((CH, H), jnp.float32),
            pltpu.VMEM_SHARED((N_PAD, H), jnp.float32),
            pltpu.SemaphoreType.DMA,
            pltpu.SemaphoreType.DMA,
        ],
    )
    def k(table_hbm, src_hbm, dst_hbm, z_hbm, out_hbm, sidx, didx, rows0,
          rows1, agg, sem0, sem1):
        cid = lax.axis_index("c")
        sid = lax.axis_index("s")
        wid = sid * NC + cid

        HC = NCHUNK // 2
        pltpu.async_copy(z_hbm, agg.at[pl.ds(sid * ZROWS, ZROWS)], sem0)
        pltpu.async_copy(src_hbm.at[pl.ds(wid * NCHUNK, HC)], sidx, sem1)
        pltpu.async_copy(dst_hbm.at[pl.ds(wid * NCHUNK, HC)], didx, sem1)
        pltpu.make_async_copy(z_hbm, agg.at[pl.ds(0, ZROWS)], sem0).wait()
        pltpu.make_async_copy(src_hbm.at[pl.ds(0, HC)], sidx, sem1).wait()
        pltpu.make_async_copy(dst_hbm.at[pl.ds(0, HC)], didx, sem1).wait()

        plsc.subcore_barrier()

        for h in range(2):
            if h:
                pltpu.sync_copy(src_hbm.at[pl.ds(wid * NCHUNK + h * HC, HC)],
                                sidx)
                pltpu.sync_copy(dst_hbm.at[pl.ds(wid * NCHUNK + h * HC, HC)],
                                didx)
            pltpu.async_copy(table_hbm.at[sidx.at[0]], rows0, sem0)

            @pl.loop(0, HC, step=2)
            def _(j):
                pltpu.make_async_copy(table_hbm.at[sidx.at[0]], rows0,
                                      sem0).wait()
                pltpu.async_copy(table_hbm.at[sidx.at[j + 1]], rows1, sem1)
                pltpu.sync_copy(rows0, agg.at[didx.at[j]], add=True)

                @pl.when(j + 2 < HC)
                def _():
                    pltpu.async_copy(table_hbm.at[sidx.at[j + 2]], rows0, sem0)

                pltpu.make_async_copy(table_hbm.at[sidx.at[0]], rows1,
                                      sem1).wait()
                pltpu.sync_copy(rows1, agg.at[didx.at[j + 1]], add=True)

        plsc.subcore_barrier()

        pltpu.sync_copy(
            agg.at[pl.ds(sid * ZROWS, ZROWS)],
            out_hbm.at[cid, pl.ds(sid * ZROWS, ZROWS)],
        )

    return k(table, src2d, dst2d, zrows)


def _gin_mlp_body(x_ref, agg_ref, w1_ref, b1_ref, w2_ref, b2_ref, o_ref):
    h = x_ref[...] + agg_ref[0] + agg_ref[1]
    t = jnp.dot(h, w1_ref[...], preferred_element_type=jnp.float32) + b1_ref[...]
    t = jnp.maximum(t, 0.0)
    o_ref[...] = jnp.dot(t, w2_ref[...], preferred_element_type=jnp.float32) + b2_ref[...]


def _gin_mlp(x, agg, W1, b1, W2, b2):
    return pl.pallas_call(
        _gin_mlp_body,
        grid=(N // R,),
        in_specs=[
            pl.BlockSpec((R, D), lambda i: (i, 0)),
            pl.BlockSpec((NC, R, H), lambda i: (0, i, 0)),
            pl.BlockSpec((D, H), lambda i: (0, 0)),
            pl.BlockSpec((1, H), lambda i: (0, 0)),
            pl.BlockSpec((H, H), lambda i: (0, 0)),
            pl.BlockSpec((1, H), lambda i: (0, 0)),
        ],
        out_specs=pl.BlockSpec((R, H), lambda i: (i, 0)),
        out_shape=jax.ShapeDtypeStruct((N, H), jnp.float32),
    )(x, agg, W1, b1.reshape(1, H), W2, b2.reshape(1, H))


def _final_body(x_ref, agg_ref, w1_ref, b1_ref, w2_ref, b2_ref, wp_ref,
                wpfull_ref, wf1a_ref, wf1b_ref, bf1_ref, wf2_ref, bf2_ref,
                o_ref, acc, zm, zacc):
    i = pl.program_id(0)

    @pl.when(i == 0)
    def _():
        acc[...] = jnp.zeros_like(acc)
        wfull = wpfull_ref[...]
        m = jnp.max(wfull, axis=0, keepdims=True)
        zm[...] = m
        zacc[...] = jnp.sum(jnp.exp(wfull - m), axis=0, keepdims=True)

    h = x_ref[...] + agg_ref[0] + agg_ref[1]
    t = jnp.dot(h, w1_ref[...], preferred_element_type=jnp.float32) + b1_ref[...]
    t = jnp.maximum(t, 0.0)
    h2 = jnp.dot(t, w2_ref[...], preferred_element_type=jnp.float32) + b2_ref[...]

    sw = jnp.exp(wp_ref[...] - zm[...]) / zacc[...]
    swb = sw.astype(jnp.bfloat16).astype(jnp.float32)
    h2b = h2.astype(jnp.bfloat16).astype(jnp.float32)
    acc[0:1, :] += jnp.sum(h2b * swb[:, 0:1], axis=0, keepdims=True)
    acc[1:2, :] += jnp.sum(h2b * swb[:, 1:2], axis=0, keepdims=True)

    @pl.when(i == (N // R) - 1)
    def _():
        g0 = acc[0:1, :]
        g1 = acc[1:2, :]
        t1 = (jnp.dot(g0, wf1a_ref[...], preferred_element_type=jnp.float32)
              + jnp.dot(g1, wf1b_ref[...], preferred_element_type=jnp.float32)
              + bf1_ref[...])
        t1 = jnp.maximum(t1, 0.0)
        o = jnp.sum(t1 * wf2_ref[...], axis=1, keepdims=True) + bf2_ref[...]
        o_ref[...] = o


def _final(x, agg, W1, b1, W2, b2, w_pool, Wf1, bf1, Wf2, bf2):
    return pl.pallas_call(
        _final_body,
        grid=(N // R,),
        in_specs=[
            pl.BlockSpec((R, D), lambda i: (i, 0)),
            pl.BlockSpec((NC, R, H), lambda i: (0, i, 0)),
            pl.BlockSpec((D, H), lambda i: (0, 0)),
            pl.BlockSpec((1, H), lambda i: (0, 0)),
            pl.BlockSpec((H, H), lambda i: (0, 0)),
            pl.BlockSpec((1, H), lambda i: (0, 0)),
            pl.BlockSpec((R, K), lambda i: (i, 0)),
            pl.BlockSpec((N, K), lambda i: (0, 0)),
            pl.BlockSpec((H, H), lambda i: (0, 0)),
            pl.BlockSpec((H, H), lambda i: (0, 0)),
            pl.BlockSpec((1, H), lambda i: (0, 0)),
            pl.BlockSpec((1, H), lambda i: (0, 0)),
            pl.BlockSpec((1, 1), lambda i: (0, 0)),
        ],
        out_specs=pl.BlockSpec((1, 1), lambda i: (0, 0)),
        out_shape=jax.ShapeDtypeStruct((1, 1), jnp.float32),
        scratch_shapes=[
            pltpu.VMEM((2, H), jnp.float32),
            pltpu.VMEM((1, K), jnp.float32),
            pltpu.VMEM((1, K), jnp.float32),
        ],
    )(x, agg, W1, b1.reshape(1, H), W2, b2.reshape(1, H), w_pool, w_pool,
      Wf1[:H], Wf1[H:], bf1.reshape(1, H), Wf2.reshape(1, H),
      bf2.reshape(1, 1))


def kernel(x, edge_index, batch, W1_0, b1_0, W2_0, b2_0, W1_1, b1_1,
           W2_1, b2_1, w_pool, Wf1, bf1, Wf2, bf2):
    src = edge_index[0]
    dst = edge_index[1]
    pad = E_PAD - E
    ar = jnp.arange(pad, dtype=jnp.int32)
    src_p = jnp.concatenate([src, ar % N])
    dst_p = jnp.concatenate([dst, N + ar % (N_PAD - N)])
    src2d = src_p.reshape(NW * NCHUNK, CH)
    dst2d = dst_p.reshape(NW * NCHUNK, CH)
    zrows = jnp.zeros((ZROWS, H), jnp.float32)

    agg0 = _seg_sum(x, src2d, dst2d, zrows)
    h1 = _gin_mlp(x, agg0, W1_0, b1_0, W2_0, b2_0)
    agg1 = _seg_sum(h1, src2d, dst2d, zrows)
    return _final(h1, agg1, W1_1, b1_1, W2_1, b2_1, w_pool, Wf1, bf1,
                  Wf2, bf2)


# speedup vs baseline: 1.0494x; 1.0494x over previous
import functools

import jax
import jax.numpy as jnp
from jax import lax
from jax.experimental import pallas as pl
from jax.experimental.pallas import tpu as pltpu
from jax.experimental.pallas import tpu_sc as plsc

N = 10000
E = 320000
D = 128
H = 128
K = 2

NC = 2
NS = 16
NW = NC * NS
CH = 128
NCHUNK = 80
EPW = NCHUNK * CH
E_PAD = NW * EPW
N_PAD = 10240
ZROWS = N_PAD // NS
R = 1000


def _seg_sum(table, src2d, dst2d):
    mesh = plsc.VectorSubcoreMesh(core_axis_name="c", subcore_axis_name="s")

    @functools.partial(
        pl.kernel,
        mesh=mesh,
        out_type=jax.ShapeDtypeStruct((NC, N_PAD, H), jnp.float32),
        scratch_types=[
            pltpu.VMEM((NCHUNK // 2, CH), jnp.int32),
            pltpu.VMEM((NCHUNK // 2, CH), jnp.int32),
            pltpu.VMEM((CH, H), jnp.float32),
            pltpu.VMEM((CH, H), jnp.float32),
            pltpu.VMEM_SHARED((N_PAD, H), jnp.float32),
            pltpu.SemaphoreType.DMA,
            pltpu.SemaphoreType.DMA,
        ],
    )
    def k(table_hbm, src_hbm, dst_hbm, out_hbm, sidx, didx, rows0,
          rows1, agg, sem0, sem1):
        cid = lax.axis_index("c")
        sid = lax.axis_index("s")
        wid = sid * NC + cid

        @pl.loop(0, CH)
        def _(i):
            @pl.loop(0, H, step=16)
            def _(l):
                rows0[i, pl.ds(l, 16)] = jnp.zeros((16,), jnp.float32)

        HC = NCHUNK // 2
        for t in range(ZROWS // CH):
            pltpu.async_copy(rows0, agg.at[pl.ds(sid * ZROWS + t * CH, CH)],
                             sem0)
        pltpu.async_copy(src_hbm.at[pl.ds(wid * NCHUNK, HC)], sidx, sem1)
        pltpu.async_copy(dst_hbm.at[pl.ds(wid * NCHUNK, HC)], didx, sem1)
        for t in range(ZROWS // CH):
            pltpu.make_async_copy(rows0, agg.at[pl.ds(0, CH)], sem0).wait()
        pltpu.make_async_copy(src_hbm.at[pl.ds(0, HC)], sidx, sem1).wait()
        pltpu.make_async_copy(dst_hbm.at[pl.ds(0, HC)], didx, sem1).wait()

        plsc.subcore_barrier()

        for h in range(2):
            if h:
                pltpu.sync_copy(src_hbm.at[pl.ds(wid * NCHUNK + h * HC, HC)],
                                sidx)
                pltpu.sync_copy(dst_hbm.at[pl.ds(wid * NCHUNK + h * HC, HC)],
                                didx)
            pltpu.async_copy(table_hbm.at[sidx.at[0]], rows0, sem0)

            @pl.loop(0, HC, step=2)
            def _(j):
                pltpu.make_async_copy(table_hbm.at[sidx.at[0]], rows0,
                                      sem0).wait()
                pltpu.async_copy(table_hbm.at[sidx.at[j + 1]], rows1, sem1)
                pltpu.sync_copy(rows0, agg.at[didx.at[j]], add=True)

                @pl.when(j + 2 < HC)
                def _():
                    pltpu.async_copy(table_hbm.at[sidx.at[j + 2]], rows0, sem0)

                pltpu.make_async_copy(table_hbm.at[sidx.at[0]], rows1,
                                      sem1).wait()
                pltpu.sync_copy(rows1, agg.at[didx.at[j + 1]], add=True)

        plsc.subcore_barrier()

        pltpu.sync_copy(
            agg.at[pl.ds(sid * ZROWS, ZROWS)],
            out_hbm.at[cid, pl.ds(sid * ZROWS, ZROWS)],
        )

    return k(table, src2d, dst2d)


def _gin_mlp_body(x_ref, agg_ref, w1_ref, b1_ref, w2_ref, b2_ref, o_ref):
    h = x_ref[...] + agg_ref[0] + agg_ref[1]
    t = jnp.dot(h, w1_ref[...], preferred_element_type=jnp.float32) + b1_ref[...]
    t = jnp.maximum(t, 0.0)
    o_ref[...] = jnp.dot(t, w2_ref[...], preferred_element_type=jnp.float32) + b2_ref[...]


def _gin_mlp(x, agg, W1, b1, W2, b2):
    return pl.pallas_call(
        _gin_mlp_body,
        grid=(N // R,),
        in_specs=[
            pl.BlockSpec((R, D), lambda i: (i, 0)),
            pl.BlockSpec((NC, R, H), lambda i: (0, i, 0)),
            pl.BlockSpec((D, H), lambda i: (0, 0)),
            pl.BlockSpec((1, H), lambda i: (0, 0)),
            pl.BlockSpec((H, H), lambda i: (0, 0)),
            pl.BlockSpec((1, H), lambda i: (0, 0)),
        ],
        out_specs=pl.BlockSpec((R, H), lambda i: (i, 0)),
        out_shape=jax.ShapeDtypeStruct((N, H), jnp.float32),
    )(x, agg, W1, b1.reshape(1, H), W2, b2.reshape(1, H))


def _final_body(x_ref, agg_ref, w1_ref, b1_ref, w2_ref, b2_ref, wp_ref,
                wpfull_ref, wf1a_ref, wf1b_ref, bf1_ref, wf2_ref, bf2_ref,
                o_ref, acc, zm, zacc):
    i = pl.program_id(0)

    @pl.when(i == 0)
    def _():
        acc[...] = jnp.zeros_like(acc)
        wfull = wpfull_ref[...]
        m = jnp.max(wfull, axis=0, keepdims=True)
        zm[...] = m
        zacc[...] = jnp.sum(jnp.exp(wfull - m), axis=0, keepdims=True)

    h = x_ref[...] + agg_ref[0] + agg_ref[1]
    t = jnp.dot(h, w1_ref[...], preferred_element_type=jnp.float32) + b1_ref[...]
    t = jnp.maximum(t, 0.0)
    h2 = jnp.dot(t, w2_ref[...], preferred_element_type=jnp.float32) + b2_ref[...]

    sw = jnp.exp(wp_ref[...] - zm[...]) / zacc[...]
    swb = sw.astype(jnp.bfloat16).astype(jnp.float32)
    h2b = h2.astype(jnp.bfloat16).astype(jnp.float32)
    acc[0:1, :] += jnp.sum(h2b * swb[:, 0:1], axis=0, keepdims=True)
    acc[1:2, :] += jnp.sum(h2b * swb[:, 1:2], axis=0, keepdims=True)

    @pl.when(i == (N // R) - 1)
    def _():
        g0 = acc[0:1, :]
        g1 = acc[1:2, :]
        t1 = (jnp.dot(g0, wf1a_ref[...], preferred_element_type=jnp.float32)
              + jnp.dot(g1, wf1b_ref[...], preferred_element_type=jnp.float32)
              + bf1_ref[...])
        t1 = jnp.maximum(t1, 0.0)
        o = jnp.sum(t1 * wf2_ref[...], axis=1, keepdims=True) + bf2_ref[...]
        o_ref[...] = o


def _final(x, agg, W1, b1, W2, b2, w_pool, Wf1, bf1, Wf2, bf2):
    return pl.pallas_call(
        _final_body,
        grid=(N // R,),
        in_specs=[
            pl.BlockSpec((R, D), lambda i: (i, 0)),
            pl.BlockSpec((NC, R, H), lambda i: (0, i, 0)),
            pl.BlockSpec((D, H), lambda i: (0, 0)),
            pl.BlockSpec((1, H), lambda i: (0, 0)),
            pl.BlockSpec((H, H), lambda i: (0, 0)),
            pl.BlockSpec((1, H), lambda i: (0, 0)),
            pl.BlockSpec((R, K), lambda i: (i, 0)),
            pl.BlockSpec((N, K), lambda i: (0, 0)),
            pl.BlockSpec((H, H), lambda i: (0, 0)),
            pl.BlockSpec((H, H), lambda i: (0, 0)),
            pl.BlockSpec((1, H), lambda i: (0, 0)),
            pl.BlockSpec((1, H), lambda i: (0, 0)),
            pl.BlockSpec((1, 1), lambda i: (0, 0)),
        ],
        out_specs=pl.BlockSpec((1, 1), lambda i: (0, 0)),
        out_shape=jax.ShapeDtypeStruct((1, 1), jnp.float32),
        scratch_shapes=[
            pltpu.VMEM((2, H), jnp.float32),
            pltpu.VMEM((1, K), jnp.float32),
            pltpu.VMEM((1, K), jnp.float32),
        ],
    )(x, agg, W1, b1.reshape(1, H), W2, b2.reshape(1, H), w_pool, w_pool,
      Wf1[:H], Wf1[H:], bf1.reshape(1, H), Wf2.reshape(1, H),
      bf2.reshape(1, 1))


def kernel(x, edge_index, batch, W1_0, b1_0, W2_0, b2_0, W1_1, b1_1,
           W2_1, b2_1, w_pool, Wf1, bf1, Wf2, bf2):
    src = edge_index[0]
    dst = edge_index[1]
    pad = E_PAD - E
    ar = jnp.arange(pad, dtype=jnp.int32)
    src_p = jnp.concatenate([src, ar % N])
    dst_p = jnp.concatenate([dst, N + ar % (N_PAD - N)])
    src2d = src_p.reshape(NW * NCHUNK, CH)
    dst2d = dst_p.reshape(NW * NCHUNK, CH)

    agg0 = _seg_sum(x, src2d, dst2d)
    h1 = _gin_mlp(x, agg0, W1_0, b1_0, W2_0, b2_0)
    agg1 = _seg_sum(h1, src2d, dst2d)
    return _final(h1, agg1, W1_1, b1_1, W2_1, b2_1, w_pool, Wf1, bf1,
                  Wf2, bf2)


# speedup vs baseline: 1.1086x; 1.0564x over previous
import functools

import numpy as np

import jax
import jax.numpy as jnp
from jax import lax
from jax.experimental import pallas as pl
from jax.experimental.pallas import tpu as pltpu
from jax.experimental.pallas import tpu_sc as plsc

N = 10000
E = 320000
D = 128
H = 128
K = 2

NC = 2
NS = 16
NW = NC * NS
CH = 128
NCHUNK = 80
EPW = NCHUNK * CH
E_PAD = NW * EPW
N_PAD = 10240
ZROWS = N_PAD // NS
R = 2000

_PAD_EDGES = np.stack([
    np.arange(E_PAD - E, dtype=np.int32) % N,
    N + np.arange(E_PAD - E, dtype=np.int32) % (N_PAD - N),
])


def _seg_sum(table, edges3):
    mesh = plsc.VectorSubcoreMesh(core_axis_name="c", subcore_axis_name="s")

    @functools.partial(
        pl.kernel,
        mesh=mesh,
        out_type=jax.ShapeDtypeStruct((NC, N_PAD, H), jnp.float32),
        scratch_types=[
            pltpu.VMEM((NCHUNK // 2, CH), jnp.int32),
            pltpu.VMEM((NCHUNK // 2, CH), jnp.int32),
            pltpu.VMEM((CH, H), jnp.float32),
            pltpu.VMEM((CH, H), jnp.float32),
            pltpu.VMEM_SHARED((N_PAD, H), jnp.float32),
            pltpu.SemaphoreType.DMA,
            pltpu.SemaphoreType.DMA,
        ],
    )
    def k(table_hbm, e_hbm, out_hbm, sidx, didx, rows0,
          rows1, agg, sem0, sem1):
        cid = lax.axis_index("c")
        sid = lax.axis_index("s")
        wid = sid * NC + cid

        @pl.loop(0, CH)
        def _(i):
            @pl.loop(0, H, step=16)
            def _(l):
                rows0[i, pl.ds(l, 16)] = jnp.zeros((16,), jnp.float32)

        HC = NCHUNK // 2
        for t in range(ZROWS // CH):
            pltpu.async_copy(rows0, agg.at[pl.ds(sid * ZROWS + t * CH, CH)],
                             sem0)
        pltpu.async_copy(e_hbm.at[0, pl.ds(wid * NCHUNK, HC)], sidx, sem1)
        pltpu.async_copy(e_hbm.at[1, pl.ds(wid * NCHUNK, HC)], didx, sem1)
        for t in range(ZROWS // CH):
            pltpu.make_async_copy(rows0, agg.at[pl.ds(0, CH)], sem0).wait()
        pltpu.make_async_copy(e_hbm.at[0, pl.ds(0, HC)], sidx, sem1).wait()
        pltpu.make_async_copy(e_hbm.at[1, pl.ds(0, HC)], didx, sem1).wait()

        plsc.subcore_barrier()

        for h in range(2):
            if h:
                pltpu.sync_copy(e_hbm.at[0, pl.ds(wid * NCHUNK + h * HC, HC)],
                                sidx)
                pltpu.sync_copy(e_hbm.at[1, pl.ds(wid * NCHUNK + h * HC, HC)],
                                didx)
            pltpu.async_copy(table_hbm.at[sidx.at[0]], rows0, sem0)

            @pl.loop(0, HC, step=2)
            def _(j):
                pltpu.make_async_copy(table_hbm.at[sidx.at[0]], rows0,
                                      sem0).wait()
                pltpu.async_copy(table_hbm.at[sidx.at[j + 1]], rows1, sem1)
                pltpu.sync_copy(rows0, agg.at[didx.at[j]], add=True)

                @pl.when(j + 2 < HC)
                def _():
                    pltpu.async_copy(table_hbm.at[sidx.at[j + 2]], rows0, sem0)

                pltpu.make_async_copy(table_hbm.at[sidx.at[0]], rows1,
                                      sem1).wait()
                pltpu.sync_copy(rows1, agg.at[didx.at[j + 1]], add=True)

        plsc.subcore_barrier()

        pltpu.sync_copy(
            agg.at[pl.ds(sid * ZROWS, ZROWS)],
            out_hbm.at[cid, pl.ds(sid * ZROWS, ZROWS)],
        )

    return k(table, edges3)


def _gin_mlp_body(x_ref, agg_ref, w1_ref, b1_ref, w2_ref, b2_ref, o_ref):
    h = x_ref[...] + agg_ref[0] + agg_ref[1]
    t = jnp.dot(h, w1_ref[...], preferred_element_type=jnp.float32) + b1_ref[...]
    t = jnp.maximum(t, 0.0)
    o_ref[...] = jnp.dot(t, w2_ref[...], preferred_element_type=jnp.float32) + b2_ref[...]


def _gin_mlp(x, agg, W1, b1, W2, b2):
    return pl.pallas_call(
        _gin_mlp_body,
        grid=(N // R,),
        in_specs=[
            pl.BlockSpec((R, D), lambda i: (i, 0)),
            pl.BlockSpec((NC, R, H), lambda i: (0, i, 0)),
            pl.BlockSpec((D, H), lambda i: (0, 0)),
            pl.BlockSpec((1, H), lambda i: (0, 0)),
            pl.BlockSpec((H, H), lambda i: (0, 0)),
            pl.BlockSpec((1, H), lambda i: (0, 0)),
        ],
        out_specs=pl.BlockSpec((R, H), lambda i: (i, 0)),
        out_shape=jax.ShapeDtypeStruct((N, H), jnp.float32),
    )(x, agg, W1, b1.reshape(1, H), W2, b2.reshape(1, H))


def _final_body(x_ref, agg_ref, w1_ref, b1_ref, w2_ref, b2_ref, wp_ref,
                wpfull_ref, wf1a_ref, wf1b_ref, bf1_ref, wf2_ref, bf2_ref,
                o_ref, acc, zm, zacc):
    i = pl.program_id(0)

    @pl.when(i == 0)
    def _():
        acc[...] = jnp.zeros_like(acc)
        wfull = wpfull_ref[...]
        m = jnp.max(wfull, axis=0, keepdims=True)
        zm[...] = m
        zacc[...] = jnp.sum(jnp.exp(wfull - m), axis=0, keepdims=True)

    h = x_ref[...] + agg_ref[0] + agg_ref[1]
    t = jnp.dot(h, w1_ref[...], preferred_element_type=jnp.float32) + b1_ref[...]
    t = jnp.maximum(t, 0.0)
    h2 = jnp.dot(t, w2_ref[...], preferred_element_type=jnp.float32) + b2_ref[...]

    sw = jnp.exp(wp_ref[...] - zm[...]) / zacc[...]
    swb = sw.astype(jnp.bfloat16).astype(jnp.float32)
    h2b = h2.astype(jnp.bfloat16).astype(jnp.float32)
    acc[0:1, :] += jnp.sum(h2b * swb[:, 0:1], axis=0, keepdims=True)
    acc[1:2, :] += jnp.sum(h2b * swb[:, 1:2], axis=0, keepdims=True)

    @pl.when(i == (N // R) - 1)
    def _():
        g0 = acc[0:1, :]
        g1 = acc[1:2, :]
        t1 = (jnp.dot(g0, wf1a_ref[...], preferred_element_type=jnp.float32)
              + jnp.dot(g1, wf1b_ref[...], preferred_element_type=jnp.float32)
              + bf1_ref[...])
        t1 = jnp.maximum(t1, 0.0)
        o = jnp.sum(t1 * wf2_ref[...], axis=1, keepdims=True) + bf2_ref[...]
        o_ref[...] = o


def _final(x, agg, W1, b1, W2, b2, w_pool, Wf1, bf1, Wf2, bf2):
    return pl.pallas_call(
        _final_body,
        grid=(N // R,),
        in_specs=[
            pl.BlockSpec((R, D), lambda i: (i, 0)),
            pl.BlockSpec((NC, R, H), lambda i: (0, i, 0)),
            pl.BlockSpec((D, H), lambda i: (0, 0)),
            pl.BlockSpec((1, H), lambda i: (0, 0)),
            pl.BlockSpec((H, H), lambda i: (0, 0)),
            pl.BlockSpec((1, H), lambda i: (0, 0)),
            pl.BlockSpec((R, K), lambda i: (i, 0)),
            pl.BlockSpec((N, K), lambda i: (0, 0)),
            pl.BlockSpec((H, H), lambda i: (0, 0)),
            pl.BlockSpec((H, H), lambda i: (1, 0)),
            pl.BlockSpec((1, H), lambda i: (0, 0)),
            pl.BlockSpec((1, H), lambda i: (0, 0)),
            pl.BlockSpec((1, 1), lambda i: (0, 0)),
        ],
        out_specs=pl.BlockSpec((1, 1), lambda i: (0, 0)),
        out_shape=jax.ShapeDtypeStruct((1, 1), jnp.float32),
        scratch_shapes=[
            pltpu.VMEM((2, H), jnp.float32),
            pltpu.VMEM((1, K), jnp.float32),
            pltpu.VMEM((1, K), jnp.float32),
        ],
    )(x, agg, W1, b1.reshape(1, H), W2, b2.reshape(1, H), w_pool, w_pool,
      Wf1, Wf1, bf1.reshape(1, H), Wf2.reshape(1, H), bf2.reshape(1, 1))


def kernel(x, edge_index, batch, W1_0, b1_0, W2_0, b2_0, W1_1, b1_1,
           W2_1, b2_1, w_pool, Wf1, bf1, Wf2, bf2):
    edges3 = jnp.concatenate(
        [edge_index, jnp.asarray(_PAD_EDGES)], axis=1,
    ).reshape(2, NW * NCHUNK, CH)

    agg0 = _seg_sum(x, edges3)
    h1 = _gin_mlp(x, agg0, W1_0, b1_0, W2_0, b2_0)
    agg1 = _seg_sum(h1, edges3)
    return _final(h1, agg1, W1_1, b1_1, W2_1, b2_1, w_pool, Wf1, bf1,
                  Wf2, bf2)
